# baseline (speedup 1.0000x reference)
import numpy as np
import jax
import jax.numpy as jnp
from jax import lax
from jax.experimental import pallas as pl
from jax.experimental.pallas import tpu as pltpu
from jax.experimental.pallas import tpu_sc as plsc

_ALPHA = 0.1
_THETA = 0.5
_NC = 2
_NS = 16
_NW = _NC * _NS
_C = 64
_NB = 4


def _make_spmm(n, n_pad, d, e_pad):
    t_chunks = e_pad // (_NW * _C)
    assert t_chunks % _NB == 0 and t_chunks >= 2 * _NB
    rpt = n_pad // _NS
    mesh = plsc.VectorSubcoreMesh(core_axis_name="c", subcore_axis_name="s",
                                  num_cores=_NC, num_subcores=_NS)

    def body(hs_hbm, pk_hbm, ewf_hbm, zeros_hbm, out_hbm, *refs):
        pk = refs[0:_NB]
        ewb = refs[_NB:2 * _NB]
        rows = refs[2 * _NB:3 * _NB]
        acc_sh = refs[3 * _NB]
        sis = refs[3 * _NB + 1:3 * _NB + 1 + _NB]
        sgs = refs[3 * _NB + 1 + _NB:3 * _NB + 1 + 2 * _NB]
        sss = refs[3 * _NB + 1 + 2 * _NB:3 * _NB + 1 + 3 * _NB]

        c = lax.axis_index("c")
        s = lax.axis_index("s")
        pltpu.sync_copy(zeros_hbm, rows[0])
        r0 = s * rpt

        def zc(b, carry):
            pltpu.sync_copy(rows[0], acc_sh.at[pl.ds(r0 + b * _C, _C)])
            return carry

        lax.fori_loop(0, rpt // _C, zc, 0)
        plsc.subcore_barrier()

        wid = c * _NS + s

        def fire_idx(t, b):
            pltpu.async_copy(pk_hbm.at[wid, t], pk[b], sis[b])
            pltpu.async_copy(ewf_hbm.at[wid, t], ewb[b], sis[b])

        def wait_idx(t, b):
            pltpu.make_async_copy(pk_hbm.at[wid, t], pk[b], sis[b]).wait()
            pltpu.make_async_copy(ewf_hbm.at[wid, t], ewb[b], sis[b]).wait()

        def fire_gather(b):
            pltpu.async_copy(hs_hbm.at[pk[b].at[0]], rows[b], sgs[b])

        def wait_gather(b):
            pltpu.make_async_copy(hs_hbm.at[pk[b].at[0]], rows[b],
                                  sgs[b]).wait()

        def fire_scatter(b):
            pltpu.async_copy(rows[b], acc_sh.at[pk[b].at[1]], sss[b],
                             add=True)

        def wait_scatter(b):
            pltpu.make_async_copy(rows[b], acc_sh.at[pk[b].at[1]],
                                  sss[b]).wait()

        def scale(b):
            rv = rows[b]

            def edge_grp(g, cy):
                wv = ewb[b][pl.ds(g * 16, 16)]
                for j in range(16):
                    w = wv[j]
                    row = g * 16 + j
                    for k in range(d // 16):
                        sl = pl.ds(k * 16, 16)
                        rv[row, sl] = rv[row, sl] * w
                return cy

            lax.fori_loop(0, _C // 16, edge_grp, 0)

        for t in range(_NB - 1):
            fire_idx(t, t)
        for t in range(_NB - 2):
            wait_idx(t, t)
            fire_gather(t)

        def blk(q, carry):
            for u in range(_NB):
                t = _NB * q + u

                @pl.when(t >= 1)
                def _():
                    wait_scatter((u + _NB - 1) % _NB)

                @pl.when(t + _NB - 1 < t_chunks)
                def _():
                    fire_idx(t + _NB - 1, (u + _NB - 1) % _NB)

                @pl.when(t + _NB - 2 < t_chunks)
                def _():
                    wait_idx(t + _NB - 2, (u + _NB - 2) % _NB)
                    fire_gather((u + _NB - 2) % _NB)

                wait_gather(u)
                scale(u)
                fire_scatter(u)
            return carry

        lax.fori_loop(0, t_chunks // _NB, blk, 0)
        wait_scatter((t_chunks - 1) % _NB)
        plsc.subcore_barrier()

        def oc(b, carry):
            sl = pl.ds(r0 + b * 128, 128)
            pltpu.sync_copy(acc_sh.at[sl], out_hbm.at[c, sl])
            return carry

        lax.fori_loop(0, rpt // 128, oc, 0)

    return pl.kernel(
        body,
        out_type=jax.ShapeDtypeStruct((_NC, n_pad, d), jnp.float32),
        mesh=mesh,
        scratch_types=(
            [pltpu.VMEM((2, _C), jnp.int32) for _ in range(_NB)]
            + [pltpu.VMEM((_C,), jnp.float32) for _ in range(_NB)]
            + [pltpu.VMEM((_C, d), jnp.float32) for _ in range(_NB)]
            + [pltpu.VMEM_SHARED((n_pad, d), jnp.float32)]
            + [pltpu.SemaphoreType.DMA for _ in range(3 * _NB)]
        ),
    )


def _make_deg1d(n_pad, e_pad):
    t_chunks = e_pad // (_NW * _C)
    rpt = n_pad // _NS
    mesh = plsc.VectorSubcoreMesh(core_axis_name="c", subcore_axis_name="s",
                                  num_cores=_NC, num_subcores=_NS)

    def body(dst_hbm, ew_hbm, zeros_hbm, out_hbm, dst_v, ew_v, acc_sh):
        c = lax.axis_index("c")
        s = lax.axis_index("s")
        wid = c * _NS + s
        r0 = s * rpt
        pltpu.sync_copy(zeros_hbm.at[pl.ds(r0, rpt)], acc_sh.at[pl.ds(r0, rpt)])
        plsc.subcore_barrier()

        base = wid * (t_chunks * _C)

        def chunk(t, carry):
            e0 = base + t * _C
            pltpu.sync_copy(dst_hbm.at[pl.ds(e0, _C)], dst_v)
            pltpu.sync_copy(ew_hbm.at[pl.ds(e0, _C)], ew_v)
            pltpu.sync_copy(ew_v, acc_sh.at[dst_v], add=True)
            return carry

        lax.fori_loop(0, t_chunks, chunk, 0)
        plsc.subcore_barrier()
        pltpu.sync_copy(acc_sh.at[pl.ds(r0, rpt)], out_hbm.at[c, pl.ds(r0, rpt)])

    return pl.kernel(
        body,
        out_type=jax.ShapeDtypeStruct((_NC, n_pad), jnp.float32),
        mesh=mesh,
        scratch_types=[
            pltpu.VMEM((_C,), jnp.int32),
            pltpu.VMEM((_C,), jnp.float32),
            pltpu.VMEM_SHARED((n_pad,), jnp.float32),
        ],
    )


def _split_cols(h):
    dd = h.shape[1]
    return jnp.stack([h[:, :dd // 2], h[:, dd // 2:]])


def _tc_in(x, w_in, b_in, degp, n):
    def body(x_ref, w_ref, b_ref, degp_ref, h0_ref, hs0_ref, dinv_ref):
        xw = jnp.dot(x_ref[...], w_ref[...], preferred_element_type=jnp.float32)
        h = jnp.maximum(xw + b_ref[...], 0.0)
        p = degp_ref[0, :, 0:1] + degp_ref[1, :, 0:1]
        deg = 1.0 + p[:n]
        dinv = jnp.where(deg > 0.0, lax.rsqrt(deg), 0.0)
        h0_ref[...] = h
        dinv_ref[...] = dinv
        hs0_ref[...] = h * dinv

    dhid = w_in.shape[1]
    return pl.pallas_call(
        body,
        out_shape=[
            jax.ShapeDtypeStruct((n, dhid), jnp.float32),
            jax.ShapeDtypeStruct((n, dhid), jnp.float32),
            jax.ShapeDtypeStruct((n, 1), jnp.float32),
        ],
    )(x, w_in, b_in, degp)


def _tc_layer(pp, hs, h0, dinv, w, beta, n):
    def body(pp_ref, hs_ref, h0_ref, dinv_ref, w_ref, out_ref):
        P = pp_ref[0, :n, :] + pp_ref[1, :n, :]
        dv = dinv_ref[...]
        agg = dv * (P + hs_ref[...])
        g = (1.0 - _ALPHA) * agg + _ALPHA * h0_ref[...]
        t = (1.0 - beta) * g + beta * jnp.dot(
            g, w_ref[...], preferred_element_type=jnp.float32)
        out_ref[...] = jnp.maximum(t, 0.0) * dv

    dhid = w.shape[1]
    return pl.pallas_call(
        body,
        out_shape=jax.ShapeDtypeStruct((n, dhid), jnp.float32),
    )(pp, hs, h0, dinv, w)


def _tc_final(pp, hs, h0, dinv, w, w_out, b_out, beta, n):
    def body(pp_ref, hs_ref, h0_ref, dinv_ref, w_ref, wo_ref, bo_ref, out_ref):
        P = pp_ref[0, :n, :] + pp_ref[1, :n, :]
        dv = dinv_ref[...]
        agg = dv * (P + hs_ref[...])
        g = (1.0 - _ALPHA) * agg + _ALPHA * h0_ref[...]
        t = (1.0 - beta) * g + beta * jnp.dot(
            g, w_ref[...], preferred_element_type=jnp.float32)
        h = jnp.maximum(t, 0.0)
        out_ref[...] = jnp.dot(
            h, wo_ref[...], preferred_element_type=jnp.float32) + bo_ref[...]

    dout = w_out.shape[1]
    return pl.pallas_call(
        body,
        out_shape=jax.ShapeDtypeStruct((n, dout), jnp.float32),
    )(pp, hs, h0, dinv, w, w_out, b_out)


def kernel(x, edge_index, edge_attr, W_in, b_in, W1, W2, W3, W_out, b_out):
    n, _ = x.shape
    e = edge_attr.shape[0]
    dhid = W_in.shape[1]

    src = edge_index[0]
    dst = edge_index[1]

    grp = _NW * _C * _NB
    e_pad = ((e + grp - 1) // grp) * grp
    pad = e_pad - e
    if pad:
        src = jnp.concatenate([src, jnp.zeros((pad,), src.dtype)])
        dst = jnp.concatenate([dst, jnp.zeros((pad,), dst.dtype)])
        ew = jnp.concatenate([edge_attr, jnp.zeros((pad,), edge_attr.dtype)])
    else:
        ew = edge_attr

    rpt = ((n + _NS - 1) // _NS + 127) // 128 * 128
    n_pad = _NS * rpt

    b_in2 = b_in.reshape(1, -1)
    b_out2 = b_out.reshape(1, -1)

    zeros1d = jnp.zeros((n_pad,), jnp.float32)
    zeros_d = jnp.zeros((_C, dhid), jnp.float32)
    degp = _make_deg1d(n_pad, e_pad)(dst, ew, zeros1d)
    h0, hs, dinv = _tc_in(x, W_in, b_in2, degp[:, :, None], n)

    t_chunks = e_pad // (_NW * _C)
    pk = jnp.stack([src, dst])
    pk = pk.reshape(2, _NW, t_chunks, _C).transpose(1, 2, 0, 3)
    ewf = ew.reshape(_NW, t_chunks, _C)

    spmm = _make_spmm(n, n_pad, dhid, e_pad)
    for i, W in enumerate([W1, W2, W3]):
        pp = spmm(hs, pk, ewf, zeros_d)
        beta = float(np.log(_THETA / (i + 1) + 1.0))
        if i < 2:
            hs = _tc_layer(pp, hs, h0, dinv, W, beta, n)
        else:
            out = _tc_final(pp, hs, h0, dinv, W, W_out, b_out2, beta, n)
    return out
